# Initial kernel scaffold; baseline (speedup 1.0000x reference)
import functools

import jax
import jax.numpy as jnp
from jax import lax
from jax.experimental import pallas as pl
from jax.experimental.pallas import tpu as pltpu
from jax.experimental.pallas import tpu_sc as plsc

GROUPS = 32
H = 32
KS = 15
EPS = 1e-5


def _cdiv(a, b):
    return (a + b - 1) // b


def _pick_window(m, width):
    budget = max(8, (96 * 1024) // (width * 4))
    best = 8
    for w in range(8, 513, 8):
        if m % w == 0 and w <= budget:
            best = w
    return best


def _sc_gather(table, idx, m_pad):
    c = table.shape[1]
    m = idx.shape[0]
    if m_pad != m:
        idx = jnp.concatenate([idx, jnp.zeros((m_pad - m,), idx.dtype)])
    w = _pick_window(m_pad, c)
    idx2 = idx.reshape(1, m_pad)
    mesh = plsc.VectorSubcoreMesh(core_axis_name="core", subcore_axis_name="subcore")

    @pl.kernel(out_type=jax.ShapeDtypeStruct((m_pad, c), table.dtype), mesh=mesh)
    def k(x_hbm, i_hbm, o_hbm):
        def body(i_vmem, o_vmem):
            pltpu.sync_copy(x_hbm.at[i_vmem.at[0]], o_vmem)

        pltpu.emit_pipeline(
            body,
            grid=(m_pad // w,),
            in_specs=[pl.BlockSpec((1, w), index_map=lambda i: (0, i))],
            out_specs=[pl.BlockSpec((w, c), index_map=lambda i: (i, 0))],
            core_axis_name=("core", "subcore"),
            dimension_semantics=(pltpu.PARALLEL,),
        )(i_hbm, o_hbm)

    return k(table, idx2)


def _gather(table, idx):
    m = idx.shape[0]
    m_pad = m if m % 8 == 0 else _cdiv(m, 128) * 128
    return _sc_gather(table, idx, m_pad)


def _mm(x, w, b, block=512):
    n, ci = x.shape
    co = w.shape[1]
    nb = _cdiv(n, block)

    def body(x_ref, w_ref, b_ref, y_ref, s_ref):
        i = pl.program_id(0)
        y = jnp.dot(x_ref[...], w_ref[...], preferred_element_type=jnp.float32)
        y = y + b_ref[...]
        y_ref[...] = y
        rows = lax.broadcasted_iota(jnp.int32, (block, 1), 0) + i * block
        ym = jnp.where(rows < n, y, 0.0)
        s0 = jnp.sum(ym, axis=0, keepdims=True)
        s1 = jnp.sum(ym * ym, axis=0, keepdims=True)
        s_ref[...] = jnp.concatenate(
            [s0, s1, jnp.zeros((6, co), jnp.float32)], axis=0)[None]

    y, stats = pl.pallas_call(
        body,
        grid=(nb,),
        in_specs=[
            pl.BlockSpec((block, ci), lambda i: (i, 0)),
            pl.BlockSpec((ci, co), lambda i: (0, 0)),
            pl.BlockSpec((1, co), lambda i: (0, 0)),
        ],
        out_specs=[
            pl.BlockSpec((block, co), lambda i: (i, 0)),
            pl.BlockSpec((1, 8, co), lambda i: (i, 0, 0)),
        ],
        out_shape=[
            jax.ShapeDtypeStruct((n, co), jnp.float32),
            jax.ShapeDtypeStruct((nb, 8, co), jnp.float32),
        ],
    )(x, w, b.reshape(1, co))
    return y, stats


def _gn_apply(y, stats, gamma, beta, relu, shortcut=None, pts=None, block=512):
    n, co = y.shape
    nb_s = stats.shape[0]
    nb = _cdiv(n, block)
    cpg = co // GROUPS
    denom = float(n * cpg)
    width = co + 16 if pts is not None else co

    def body(*refs):
        if pts is not None and shortcut is not None:
            y_ref, s_ref, g_ref, b_ref, sc_ref, p_ref, o_ref = refs
        elif pts is not None:
            y_ref, s_ref, g_ref, b_ref, p_ref, o_ref = refs
            sc_ref = None
        elif shortcut is not None:
            y_ref, s_ref, g_ref, b_ref, sc_ref, o_ref = refs
            p_ref = None
        else:
            y_ref, s_ref, g_ref, b_ref, o_ref = refs
            sc_ref = p_ref = None
        s = jnp.sum(s_ref[...], axis=0)
        colsum = s[0:1]
        colsq = s[1:2]
        gi = lax.broadcasted_iota(jnp.int32, (co, GROUPS), 0) // cpg
        gj = lax.broadcasted_iota(jnp.int32, (co, GROUPS), 1)
        gm = (gi == gj).astype(jnp.float32)
        mean_g = jnp.dot(colsum, gm, preferred_element_type=jnp.float32) / denom
        m2_g = jnp.dot(colsq, gm, preferred_element_type=jnp.float32) / denom
        var_g = m2_g - mean_g * mean_g
        rstd_g = lax.rsqrt(var_g + EPS)
        mean_c = jnp.dot(mean_g, gm.T, preferred_element_type=jnp.float32)
        rstd_c = jnp.dot(rstd_g, gm.T, preferred_element_type=jnp.float32)
        out = (y_ref[...] - mean_c) * rstd_c * g_ref[...] + b_ref[...]
        if sc_ref is not None:
            out = out + sc_ref[...]
        if relu:
            out = jnp.where(out >= 0, out, 0.1 * out)
        if p_ref is not None:
            rowsum = jnp.sum(out, axis=1, keepdims=True)
            o_ref[...] = jnp.concatenate(
                [out, p_ref[...], rowsum, jnp.zeros((block, 12), jnp.float32)],
                axis=1)
        else:
            o_ref[...] = out

    in_specs = [
        pl.BlockSpec((block, co), lambda i: (i, 0)),
        pl.BlockSpec((nb_s, 8, co), lambda i: (0, 0, 0)),
        pl.BlockSpec((1, co), lambda i: (0, 0)),
        pl.BlockSpec((1, co), lambda i: (0, 0)),
    ]
    args = [y, stats, gamma.reshape(1, co), beta.reshape(1, co)]
    if shortcut is not None:
        in_specs.append(pl.BlockSpec((block, co), lambda i: (i, 0)))
        args.append(shortcut)
    if pts is not None:
        in_specs.append(pl.BlockSpec((block, 3), lambda i: (i, 0)))
        args.append(pts)
    return pl.pallas_call(
        body,
        grid=(nb,),
        in_specs=in_specs,
        out_specs=pl.BlockSpec((block, width), lambda i: (i, 0)),
        out_shape=jax.ShapeDtypeStruct((n, width), jnp.float32),
    )(*args)


def _kpconv(gath, q_pts, kp_t, w_k, sigma, n, c, bq):
    d = w_k.shape[-1]
    nb = _cdiv(n, bq)
    bh = bq * H
    ctot = gath.shape[1]

    def body(g_ref, q_ref, kpt_ref, w_ref, o_ref, s_ref):
        i = pl.program_id(0)
        g = g_ref[...]
        p = g[:, c:c + 3]
        p3 = p.reshape(bq, H, 3)
        d3 = p3 - q_ref[...][:, None, :]
        dm = d3.reshape(bh, 3)
        r2 = jnp.sum(dm * dm, axis=1, keepdims=True)
        kpt = kpt_ref[...]
        kpn2 = jnp.sum(kpt * kpt, axis=0, keepdims=True)
        cross = jnp.dot(dm, kpt, preferred_element_type=jnp.float32)
        sqd = r2 + kpn2 - 2.0 * cross
        nw = jnp.maximum(1.0 - jnp.sqrt(sqd + 1e-12) / sigma, 0.0)
        nw3 = nw.reshape(bq, H, KS)
        if c == 0:
            nwsum = jnp.sum(nw3, axis=1)
            acc = jnp.dot(nwsum, w_ref[...], preferred_element_type=jnp.float32)
            out = acc / float(H)
        else:
            f3 = g[:, :c].reshape(bq, H, c)
            acc = jnp.zeros((bq, d), jnp.float32)
            for k in range(KS):
                wfk = jnp.sum(nw3[:, :, k:k + 1] * f3, axis=1)
                acc = acc + jnp.dot(wfk, w_ref[k],
                                    preferred_element_type=jnp.float32)
            rs3 = g[:, c + 3:c + 4].reshape(bq, H, 1)
            cnt = jnp.sum((rs3 > 0.0).astype(jnp.float32), axis=1)
            out = acc / jnp.maximum(cnt, 1.0)
        o_ref[...] = out
        rows = lax.broadcasted_iota(jnp.int32, (bq, 1), 0) + i * bq
        om = jnp.where(rows < n, out, 0.0)
        s0 = jnp.sum(om, axis=0, keepdims=True)
        s1 = jnp.sum(om * om, axis=0, keepdims=True)
        s_ref[...] = jnp.concatenate(
            [s0, s1, jnp.zeros((6, d), jnp.float32)], axis=0)[None]

    w_spec = (pl.BlockSpec((KS, d), lambda i: (0, 0)) if c == 0
              else pl.BlockSpec((KS, c, d), lambda i: (0, 0, 0)))
    out, stats = pl.pallas_call(
        body,
        grid=(nb,),
        in_specs=[
            pl.BlockSpec((bh, ctot), lambda i: (i, 0)),
            pl.BlockSpec((bq, 3), lambda i: (i, 0)),
            pl.BlockSpec((3, KS), lambda i: (0, 0)),
            w_spec,
        ],
        out_specs=[
            pl.BlockSpec((bq, d), lambda i: (i, 0)),
            pl.BlockSpec((1, 8, d), lambda i: (i, 0, 0)),
        ],
        out_shape=[
            jax.ShapeDtypeStruct((n, d), jnp.float32),
            jax.ShapeDtypeStruct((nb, 8, d), jnp.float32),
        ],
    )(gath, q_pts, kp_t, w_k)
    return out, stats


def _maxpool_reduce(gath, n_out, c, bq=128):
    nb = _cdiv(n_out, bq)
    bh = bq * H

    def body(g_ref, o_ref):
        g3 = g_ref[...].reshape(bq, H, c)
        o_ref[...] = jnp.max(g3, axis=1)

    return pl.pallas_call(
        body,
        grid=(nb,),
        in_specs=[pl.BlockSpec((bh, c), lambda i: (i, 0))],
        out_specs=pl.BlockSpec((bq, c), lambda i: (i, 0)),
        out_shape=jax.ShapeDtypeStruct((n_out, c), jnp.float32),
    )(gath)


def _conv_bq(c):
    if c <= 64:
        return 256
    if c <= 128:
        return 128
    return 64


def _kpconv_block(p, table, q_pts, nidx, sigma, c):
    gath = _gather(table, nidx.reshape(-1))
    kp_t = p['kp'].T
    n = q_pts.shape[0]
    w_k = p['W'][:, 0, :] if c == 0 else p['W']
    return _kpconv(gath, q_pts, kp_t, w_k, sigma, n, c, _conv_bq(max(c, 1)))


def _residual_block(p, s_feats, q_pts, s_pts, nidx, sigma, strided):
    n_src, cin = s_feats.shape
    mid = p['unary1']['W'].shape[1]
    y1, st1 = _mm(s_feats, p['unary1']['W'], p['unary1']['b'])
    table = _gn_apply(y1, st1, p['unary1']['gn']['gamma'],
                      p['unary1']['gn']['beta'], relu=True, pts=s_pts)
    cv, stc = _kpconv_block(p['conv'], table, q_pts, nidx, sigma, mid)
    x = _gn_apply(cv, stc, p['conv']['gn']['gamma'], p['conv']['gn']['beta'],
                  relu=True)
    y2, st2 = _mm(x, p['unary2']['W'], p['unary2']['b'])
    if strided:
        gath = _gather(s_feats, nidx.reshape(-1))
        shortcut = _maxpool_reduce(gath, q_pts.shape[0], cin)
    else:
        shortcut = s_feats
    if 'shortcut' in p:
        ys, sts = _mm(shortcut, p['shortcut']['W'], p['shortcut']['b'])
        shortcut = _gn_apply(ys, sts, p['shortcut']['gn']['gamma'],
                             p['shortcut']['gn']['beta'], relu=False)
    return _gn_apply(y2, st2, p['unary2']['gn']['gamma'],
                     p['unary2']['gn']['beta'], relu=True, shortcut=shortcut)


def kernel(feats, points_0, points_1, points_2, points_3,
           neighbors_0, neighbors_1, neighbors_2, neighbors_3,
           subsampling_0, subsampling_1, subsampling_2,
           upsampling_0, upsampling_1, upsampling_2, params):
    pts = [points_0, points_1, points_2, points_3]
    nbrs = [neighbors_0, neighbors_1, neighbors_2, neighbors_3]
    subs = [subsampling_0, subsampling_1, subsampling_2]
    ups = [upsampling_0, upsampling_1, upsampling_2]
    p = params
    s = 0.05

    n0 = pts[0].shape[0]
    geo0 = jnp.concatenate([pts[0], jnp.zeros((n0, 13), jnp.float32)], axis=1)
    cv0, st0 = _kpconv_block(p['b00'], geo0, pts[0], nbrs[0], s, 0)
    x = _gn_apply(cv0, st0, p['b00']['gn']['gamma'], p['b00']['gn']['beta'],
                  relu=True)

    x = _residual_block(p['b01'], x, pts[0], pts[0], nbrs[0], s, False)
    r0 = x
    x = _residual_block(p['b10'], x, pts[1], pts[0], subs[0], s, True)
    x = _residual_block(p['b11'], x, pts[1], pts[1], nbrs[1], 2 * s, False)
    x = _residual_block(p['b12'], x, pts[1], pts[1], nbrs[1], 2 * s, False)
    r1 = x
    x = _residual_block(p['b20'], x, pts[2], pts[1], subs[1], 2 * s, True)
    x = _residual_block(p['b21'], x, pts[2], pts[2], nbrs[2], 4 * s, False)
    x = _residual_block(p['b22'], x, pts[2], pts[2], nbrs[2], 4 * s, False)
    r2 = x
    x = _residual_block(p['b30'], x, pts[2 + 1], pts[2], subs[2], 4 * s, True)
    x = _residual_block(p['b31'], x, pts[3], pts[3], nbrs[3], 8 * s, False)
    x = _residual_block(p['b32'], x, pts[3], pts[3], nbrs[3], 8 * s, False)
    r3 = x

    up3 = _gather(r3, ups[2][:, 0])[:r2.shape[0]]
    l3in = jnp.concatenate([up3, r2], axis=1)
    y, st = _mm(l3in, p['last0']['W'], p['last0']['b'])
    l3 = _gn_apply(y, st, p['last0']['gn']['gamma'], p['last0']['gn']['beta'],
                   relu=True)
    up2 = _gather(l3, ups[1][:, 0])[:r1.shape[0]]
    l2in = jnp.concatenate([up2, r1], axis=1)
    l2, _ = _mm(l2in, p['last1']['W'], p['last1']['b'])
    return (l2, l3, r3)


# speedup vs baseline: 2.2575x; 2.2575x over previous
import functools

import jax
import jax.numpy as jnp
from jax import lax
from jax.experimental import pallas as pl
from jax.experimental.pallas import tpu as pltpu
from jax.experimental.pallas import tpu_sc as plsc

GROUPS = 32
H = 32
KS = 15
EPS = 1e-5


def _cdiv(a, b):
    return (a + b - 1) // b


def _sqrt_exact(x):
    r = lax.rsqrt(x)
    r = r * (1.5 - 0.5 * x * r * r)
    return x * r


def _rsqrt_exact(x):
    r = lax.rsqrt(x)
    return r * (1.5 - 0.5 * x * r * r)


def _recip_exact(x):
    q = 1.0 / x
    return q * (2.0 - x * q)


def _pick_window(m, width):
    budget = max(128, (96 * 1024) // (width * 4))
    best = 128
    for w in (256, 384, 512):
        if m % w == 0 and w <= budget:
            best = w
    return best


def _sc_gather(table, idx, m_pad):
    c = table.shape[1]
    m = idx.shape[0]
    if m_pad != m:
        idx = jnp.concatenate([idx, jnp.zeros((m_pad - m,), idx.dtype)])
    w = _pick_window(m_pad, c)
    idx2 = idx.reshape(1, m_pad)
    mesh = plsc.VectorSubcoreMesh(core_axis_name="core", subcore_axis_name="subcore")

    @pl.kernel(out_type=jax.ShapeDtypeStruct((m_pad, c), table.dtype), mesh=mesh)
    def k(x_hbm, i_hbm, o_hbm):
        def body(i_vmem, o_vmem):
            pltpu.sync_copy(x_hbm.at[i_vmem.at[0]], o_vmem)

        pltpu.emit_pipeline(
            body,
            grid=(m_pad // w,),
            in_specs=[pl.BlockSpec((1, w), index_map=lambda i: (0, i))],
            out_specs=[pl.BlockSpec((w, c), index_map=lambda i: (i, 0))],
            core_axis_name=("core", "subcore"),
            dimension_semantics=(pltpu.PARALLEL,),
        )(i_hbm, o_hbm)

    return k(table, idx2)


def _gather(table, idx):
    m = idx.shape[0]
    m_pad = _cdiv(m, 128) * 128
    c = table.shape[1]
    if c > 384:
        parts = [_sc_gather(table[:, s:s + 256], idx, m_pad)
                 for s in range(0, c, 256)]
        return jnp.concatenate(parts, axis=1)
    return _sc_gather(table, idx, m_pad)


def _mm(x, w, b, block=512):
    n, ci = x.shape
    co = w.shape[1]
    nb = _cdiv(n, block)

    def body(x_ref, w_ref, b_ref, y_ref, s_ref):
        i = pl.program_id(0)
        y = jnp.dot(x_ref[...], w_ref[...], preferred_element_type=jnp.float32)
        y = y + b_ref[...]
        y_ref[...] = y
        rows = lax.broadcasted_iota(jnp.int32, (block, 1), 0) + i * block
        ym = jnp.where(rows < n, y, 0.0)
        s0 = jnp.sum(ym, axis=0, keepdims=True)
        s1 = jnp.sum(ym * ym, axis=0, keepdims=True)
        s_ref[...] = jnp.concatenate(
            [s0, s1, jnp.zeros((6, co), jnp.float32)], axis=0)[None]

    y, stats = pl.pallas_call(
        body,
        grid=(nb,),
        in_specs=[
            pl.BlockSpec((block, ci), lambda i: (i, 0)),
            pl.BlockSpec((ci, co), lambda i: (0, 0)),
            pl.BlockSpec((1, co), lambda i: (0, 0)),
        ],
        out_specs=[
            pl.BlockSpec((block, co), lambda i: (i, 0)),
            pl.BlockSpec((1, 8, co), lambda i: (i, 0, 0)),
        ],
        out_shape=[
            jax.ShapeDtypeStruct((n, co), jnp.float32),
            jax.ShapeDtypeStruct((nb, 8, co), jnp.float32),
        ],
    )(x, w, b.reshape(1, co))
    return y, stats


def _gn_apply(y, stats, gamma, beta, relu, shortcut=None, pts=None, block=512):
    n, co = y.shape
    nb_s = stats.shape[0]
    nb = _cdiv(n, block)
    cpg = co // GROUPS
    denom = float(n * cpg)
    width = _cdiv(co + 16, 128) * 128 if pts is not None else co

    def body(*refs):
        if pts is not None and shortcut is not None:
            y_ref, s_ref, g_ref, b_ref, sc_ref, p_ref, o_ref = refs
        elif pts is not None:
            y_ref, s_ref, g_ref, b_ref, p_ref, o_ref = refs
            sc_ref = None
        elif shortcut is not None:
            y_ref, s_ref, g_ref, b_ref, sc_ref, o_ref = refs
            p_ref = None
        else:
            y_ref, s_ref, g_ref, b_ref, o_ref = refs
            sc_ref = p_ref = None
        s = jnp.sum(s_ref[...], axis=0)
        colsum = s[0:1]
        colsq = s[1:2]
        gi = lax.broadcasted_iota(jnp.int32, (co, GROUPS), 0) // cpg
        gj = lax.broadcasted_iota(jnp.int32, (co, GROUPS), 1)
        gm = (gi == gj).astype(jnp.float32)
        hi = lax.Precision.HIGHEST
        mean_g = jnp.dot(colsum, gm, preferred_element_type=jnp.float32,
                         precision=hi) / denom
        m2_g = jnp.dot(colsq, gm, preferred_element_type=jnp.float32,
                       precision=hi) / denom
        var_g = m2_g - mean_g * mean_g
        rstd_g = _rsqrt_exact(var_g + EPS)
        mean_c = jnp.dot(mean_g, gm.T, preferred_element_type=jnp.float32,
                         precision=hi)
        rstd_c = jnp.dot(rstd_g, gm.T, preferred_element_type=jnp.float32,
                         precision=hi)
        out = (y_ref[...] - mean_c) * rstd_c * g_ref[...] + b_ref[...]
        if sc_ref is not None:
            out = out + sc_ref[...]
        if relu:
            out = jnp.where(out >= 0, out, 0.1 * out)
        if p_ref is not None:
            rowsum = jnp.sum(out, axis=1, keepdims=True)
            o_ref[...] = jnp.concatenate(
                [out, p_ref[...], rowsum,
                 jnp.zeros((block, width - co - 4), jnp.float32)], axis=1)
        else:
            o_ref[...] = out

    in_specs = [
        pl.BlockSpec((block, co), lambda i: (i, 0)),
        pl.BlockSpec((nb_s, 8, co), lambda i: (0, 0, 0)),
        pl.BlockSpec((1, co), lambda i: (0, 0)),
        pl.BlockSpec((1, co), lambda i: (0, 0)),
    ]
    args = [y, stats, gamma.reshape(1, co), beta.reshape(1, co)]
    if shortcut is not None:
        in_specs.append(pl.BlockSpec((block, co), lambda i: (i, 0)))
        args.append(shortcut)
    if pts is not None:
        in_specs.append(pl.BlockSpec((block, 3), lambda i: (i, 0)))
        args.append(pts)
    return pl.pallas_call(
        body,
        grid=(nb,),
        in_specs=in_specs,
        out_specs=pl.BlockSpec((block, width), lambda i: (i, 0)),
        out_shape=jax.ShapeDtypeStruct((n, width), jnp.float32),
    )(*args)


def _kpconv(gath, q_pts, kp_t, w_k, sigma, n, c, bq):
    d = w_k.shape[-1]
    nb = _cdiv(n, bq)
    bh = bq * H
    ctot = gath.shape[1]

    def body(g_ref, q_ref, kpt_ref, w_ref, o_ref, s_ref):
        i = pl.program_id(0)
        g = g_ref[...]
        p = g[:, c:c + 3]
        p3 = p.reshape(bq, H, 3)
        d3 = p3 - q_ref[...][:, None, :]
        dm = d3.reshape(bh, 3)
        kpt = kpt_ref[...]
        dx = dm[:, 0:1] - kpt[0:1, :]
        dy = dm[:, 1:2] - kpt[1:2, :]
        dz = dm[:, 2:3] - kpt[2:3, :]
        sqd = dx * dx + dy * dy + dz * dz
        nw = jnp.maximum(1.0 - _sqrt_exact(sqd + 1e-12) * (1.0 / sigma),
                         0.0)
        nw3 = nw.reshape(bq, H, KS)
        if c == 0:
            nwsum = jnp.sum(nw3, axis=1)
            acc = jnp.dot(nwsum, w_ref[...], preferred_element_type=jnp.float32)
            out = acc / float(H)
        else:
            f3 = g[:, :c].reshape(bq, H, c)
            f3 = f3.astype(jnp.bfloat16).astype(jnp.float32)
            nw3 = nw3.astype(jnp.bfloat16).astype(jnp.float32)
            acc = jnp.zeros((bq, d), jnp.float32)
            for k in range(KS):
                wfk = jnp.sum(nw3[:, :, k:k + 1] * f3, axis=1)
                acc = acc + jnp.dot(wfk, w_ref[k],
                                    preferred_element_type=jnp.float32)
            rs3 = g[:, c + 3:c + 4].reshape(bq, H, 1)
            cnt = jnp.sum((rs3 > 0.0).astype(jnp.float32), axis=1)
            out = acc * _recip_exact(jnp.maximum(cnt, 1.0))
        o_ref[...] = out
        rows = lax.broadcasted_iota(jnp.int32, (bq, 1), 0) + i * bq
        om = jnp.where(rows < n, out, 0.0)
        s0 = jnp.sum(om, axis=0, keepdims=True)
        s1 = jnp.sum(om * om, axis=0, keepdims=True)
        s_ref[...] = jnp.concatenate(
            [s0, s1, jnp.zeros((6, d), jnp.float32)], axis=0)[None]

    w_spec = (pl.BlockSpec((KS, d), lambda i: (0, 0)) if c == 0
              else pl.BlockSpec((KS, c, d), lambda i: (0, 0, 0)))
    out, stats = pl.pallas_call(
        body,
        grid=(nb,),
        in_specs=[
            pl.BlockSpec((bh, ctot), lambda i: (i, 0)),
            pl.BlockSpec((bq, 3), lambda i: (i, 0)),
            pl.BlockSpec((3, KS), lambda i: (0, 0)),
            w_spec,
        ],
        out_specs=[
            pl.BlockSpec((bq, d), lambda i: (i, 0)),
            pl.BlockSpec((1, 8, d), lambda i: (i, 0, 0)),
        ],
        out_shape=[
            jax.ShapeDtypeStruct((n, d), jnp.float32),
            jax.ShapeDtypeStruct((nb, 8, d), jnp.float32),
        ],
    )(gath, q_pts, kp_t, w_k)
    return out, stats


def _maxpool_reduce(gath, n_out, c, bq=128):
    nb = _cdiv(n_out, bq)
    bh = bq * H

    def body(g_ref, o_ref):
        g3 = g_ref[...].reshape(bq, H, c)
        o_ref[...] = jnp.max(g3, axis=1)

    return pl.pallas_call(
        body,
        grid=(nb,),
        in_specs=[pl.BlockSpec((bh, c), lambda i: (i, 0))],
        out_specs=pl.BlockSpec((bq, c), lambda i: (i, 0)),
        out_shape=jax.ShapeDtypeStruct((n_out, c), jnp.float32),
    )(gath)


def _conv_bq(c):
    if c <= 64:
        return 256
    if c <= 128:
        return 128
    return 64


def _kpconv_block(p, table, q_pts, nidx, sigma, c):
    gath = _gather(table, nidx.reshape(-1))
    kp_t = p['kp'].T
    n = q_pts.shape[0]
    w_k = p['W'][:, 0, :] if c == 0 else p['W']
    return _kpconv(gath, q_pts, kp_t, w_k, sigma, n, c, _conv_bq(max(c, 1)))


def _residual_block(p, s_feats, q_pts, s_pts, nidx, sigma, strided):
    n_src, cin = s_feats.shape
    mid = p['unary1']['W'].shape[1]
    y1, st1 = _mm(s_feats, p['unary1']['W'], p['unary1']['b'])
    table = _gn_apply(y1, st1, p['unary1']['gn']['gamma'],
                      p['unary1']['gn']['beta'], relu=True, pts=s_pts)
    cv, stc = _kpconv_block(p['conv'], table, q_pts, nidx, sigma, mid)
    x = _gn_apply(cv, stc, p['conv']['gn']['gamma'], p['conv']['gn']['beta'],
                  relu=True)
    y2, st2 = _mm(x, p['unary2']['W'], p['unary2']['b'])
    if strided:
        gath = _gather(s_feats, nidx.reshape(-1))
        shortcut = _maxpool_reduce(gath, q_pts.shape[0], cin)
    else:
        shortcut = s_feats
    if 'shortcut' in p:
        ys, sts = _mm(shortcut, p['shortcut']['W'], p['shortcut']['b'])
        shortcut = _gn_apply(ys, sts, p['shortcut']['gn']['gamma'],
                             p['shortcut']['gn']['beta'], relu=False)
    return _gn_apply(y2, st2, p['unary2']['gn']['gamma'],
                     p['unary2']['gn']['beta'], relu=True, shortcut=shortcut)


def kernel(feats, points_0, points_1, points_2, points_3,
           neighbors_0, neighbors_1, neighbors_2, neighbors_3,
           subsampling_0, subsampling_1, subsampling_2,
           upsampling_0, upsampling_1, upsampling_2, params):
    pts = [points_0, points_1, points_2, points_3]
    nbrs = [neighbors_0, neighbors_1, neighbors_2, neighbors_3]
    subs = [subsampling_0, subsampling_1, subsampling_2]
    ups = [upsampling_0, upsampling_1, upsampling_2]
    p = params
    s = 0.05

    n0 = pts[0].shape[0]
    geo0 = jnp.concatenate([pts[0], jnp.zeros((n0, 125), jnp.float32)], axis=1)
    cv0, st0 = _kpconv_block(p['b00'], geo0, pts[0], nbrs[0], s, 0)
    x = _gn_apply(cv0, st0, p['b00']['gn']['gamma'], p['b00']['gn']['beta'],
                  relu=True)

    x = _residual_block(p['b01'], x, pts[0], pts[0], nbrs[0], s, False)
    r0 = x
    x = _residual_block(p['b10'], x, pts[1], pts[0], subs[0], s, True)
    x = _residual_block(p['b11'], x, pts[1], pts[1], nbrs[1], 2 * s, False)
    x = _residual_block(p['b12'], x, pts[1], pts[1], nbrs[1], 2 * s, False)
    r1 = x
    x = _residual_block(p['b20'], x, pts[2], pts[1], subs[1], 2 * s, True)
    x = _residual_block(p['b21'], x, pts[2], pts[2], nbrs[2], 4 * s, False)
    x = _residual_block(p['b22'], x, pts[2], pts[2], nbrs[2], 4 * s, False)
    r2 = x
    x = _residual_block(p['b30'], x, pts[2 + 1], pts[2], subs[2], 4 * s, True)
    x = _residual_block(p['b31'], x, pts[3], pts[3], nbrs[3], 8 * s, False)
    x = _residual_block(p['b32'], x, pts[3], pts[3], nbrs[3], 8 * s, False)
    r3 = x

    up3 = _gather(r3, ups[2][:, 0])[:r2.shape[0]]
    l3in = jnp.concatenate([up3, r2], axis=1)
    y, st = _mm(l3in, p['last0']['W'], p['last0']['b'])
    l3 = _gn_apply(y, st, p['last0']['gn']['gamma'], p['last0']['gn']['beta'],
                   relu=True)
    up2 = _gather(l3, ups[1][:, 0])[:r1.shape[0]]
    l2in = jnp.concatenate([up2, r1], axis=1)
    l2, _ = _mm(l2in, p['last1']['W'], p['last1']['b'])
    return (l2, l3, r3)
